# baseline (speedup 1.0000x reference)
import functools

import jax
import jax.numpy as jnp
from jax import lax
from jax.experimental import pallas as pl
from jax.experimental.pallas import tpu as pltpu
from jax.experimental.pallas import tpu_sc as plsc

_FT = 512
_NW = 32


def _safe_div(num, den):
    den = jnp.where(jnp.abs(den) < 1e-12, jnp.where(den < 0, -1e-12, 1e-12), den)
    return num / den


def _tc_body(p_ref, a_ref, b_ref, c_ref, wts_ref, hf_ref, cf_ref, nrm_ref):
    S = p_ref.shape[0]
    Fp = a_ref.shape[1]
    n_tiles = Fp // _FT

    px = p_ref[:, 0:1]
    py = p_ref[:, 1:2]
    pz = p_ref[:, 2:3]

    def tile(t, carry):
        (bd2, bu, bv, bw, bhx, bhy, bhz, bnx, bny, bnz, bf0, bf1, bf2) = carry
        ds = pl.ds(t * _FT, _FT)
        ax = a_ref[0:1, ds]; ay = a_ref[1:2, ds]; az = a_ref[2:3, ds]
        bx = b_ref[0:1, ds]; by = b_ref[1:2, ds]; bz = b_ref[2:3, ds]
        cx = c_ref[0:1, ds]; cy = c_ref[1:2, ds]; cz = c_ref[2:3, ds]
        f0 = a_ref[3:4, ds]; f1 = b_ref[3:4, ds]; f2 = c_ref[3:4, ds]

        abx = bx - ax; aby = by - ay; abz = bz - az
        acx = cx - ax; acy = cy - ay; acz = cz - az

        apx = px - ax; apy = py - ay; apz = pz - az
        d1 = (abx * apx + abz * apz) + aby * apy
        d2 = (acx * apx + acz * apz) + acy * apy
        bpx = px - bx; bpy = py - by; bpz = pz - bz
        d3 = (abx * bpx + abz * bpz) + aby * bpy
        d4 = (acx * bpx + acz * bpz) + acy * bpy
        cpx = px - cx; cpy = py - cy; cpz = pz - cz
        d5 = (abx * cpx + abz * cpz) + aby * cpy
        d6 = (acx * cpx + acz * cpz) + acy * cpy

        vc = d1 * d4 - d3 * d2
        vb = d5 * d2 - d1 * d6
        va = d3 * d6 - d5 * d4
        v_ab = _safe_div(d1, d1 - d3)
        w_ac = _safe_div(d2, d2 - d6)
        w_bc = _safe_div(d4 - d3, (d4 - d3) + (d5 - d6))
        denom = _safe_div(jnp.ones_like(va), va + vb + vc)
        v_in = vb * denom
        w_in = vc * denom
        z = jnp.zeros_like(d1)
        o = jnp.ones_like(d1)
        c1 = (d1 <= 0) & (d2 <= 0)
        c2 = (d3 >= 0) & (d4 <= d3)
        c3 = (vc <= 0) & (d1 >= 0) & (d3 <= 0)
        c4 = (d6 >= 0) & (d5 <= d6)
        c5 = (vb <= 0) & (d2 >= 0) & (d6 <= 0)
        c6 = (va <= 0) & ((d4 - d3) >= 0) & ((d5 - d6) >= 0)
        conds = [c1, c2, c3, c4, c5, c6]

        def _select(vals, default):
            out = default
            for cnd, val in reversed(list(zip(conds, vals))):
                out = jnp.where(cnd, val, out)
            return out

        u = _select([o, z, o - v_ab, z, o - w_ac, z], o - v_in - w_in)
        v = _select([z, o, v_ab, z, z, o - w_bc], v_in)
        w = _select([z, z, z, o, w_ac, w_bc], w_in)

        hx = (u * ax + v * bx) + w * cx
        hy = (u * ay + v * by) + w * cy
        hz = (u * az + v * bz) + w * cz
        dx = hx - px; dy = hy - py; dz = hz - pz
        d2t = (dx * dx + dz * dz) + dy * dy

        nxf = aby * acz - abz * acy
        nyf = abz * acx - abx * acz
        nzf = abx * acy - aby * acx

        m = jnp.min(d2t, axis=1, keepdims=True)
        lane = lax.broadcasted_iota(jnp.int32, (S, _FT), 1)
        lsel = jnp.min(jnp.where(d2t == m, lane, _FT), axis=1, keepdims=True)
        onehot = lane == lsel

        def pick(q):
            return jnp.sum(jnp.where(onehot, q, 0.0), axis=1, keepdims=True)

        tu = pick(u); tv = pick(v); tw = pick(w)
        thx = pick(hx); thy = pick(hy); thz = pick(hz)
        tnx = pick(nxf); tny = pick(nyf); tnz = pick(nzf)
        tf0 = pick(f0); tf1 = pick(f1); tf2 = pick(f2)

        better = m < bd2
        bd2 = jnp.where(better, m, bd2)
        bu = jnp.where(better, tu, bu)
        bv = jnp.where(better, tv, bv)
        bw = jnp.where(better, tw, bw)
        bhx = jnp.where(better, thx, bhx)
        bhy = jnp.where(better, thy, bhy)
        bhz = jnp.where(better, thz, bhz)
        bnx = jnp.where(better, tnx, bnx)
        bny = jnp.where(better, tny, bny)
        bnz = jnp.where(better, tnz, bnz)
        bf0 = jnp.where(better, tf0, bf0)
        bf1 = jnp.where(better, tf1, bf1)
        bf2 = jnp.where(better, tf2, bf2)
        return (bd2, bu, bv, bw, bhx, bhy, bhz, bnx, bny, bnz, bf0, bf1, bf2)

    zf = jnp.zeros((S, 1), jnp.float32)
    init = (jnp.full((S, 1), jnp.inf, jnp.float32),
            zf, zf, zf, zf, zf, zf, zf, zf, zf, zf, zf, zf)
    (bd2, bu, bv, bw, bhx, bhy, bhz, bnx, bny, bnz, bf0, bf1, bf2) = \
        lax.fori_loop(0, n_tiles, tile, init)

    sdot = ((px - bhx) * bnx + (pz - bhz) * bnz) + (py - bhy) * bny
    sgn = jnp.sign(sdot)
    dist = jnp.sqrt(jnp.maximum(bd2, 1e-12))
    sdf = dist * sgn

    dxo = bhx - px; dyo = bhy - py; dzo = bhz - pz
    nrm = jnp.maximum(jnp.sqrt((dxo * dxo + dzo * dzo) + dyo * dyo), 1e-6)

    wts_ref[:, 0:1] = bu
    wts_ref[:, 1:2] = bv
    wts_ref[:, 2:3] = bw
    hf_ref[:, 0:1] = bf0.astype(jnp.int32)
    hf_ref[:, 1:2] = bf1.astype(jnp.int32)
    hf_ref[:, 2:3] = bf2.astype(jnp.int32)
    cf_ref[:, 0:1] = bv
    cf_ref[:, 1:2] = bw
    cf_ref[:, 2:3] = sdf
    nrm_ref[:, 0:1] = dxo / nrm
    nrm_ref[:, 1:2] = dyo / nrm
    nrm_ref[:, 2:3] = dzo / nrm


def _closest_point_tc(coords, at, bt, ct):
    B, S, _ = coords.shape
    Fp = at.shape[2]
    face_spec = pl.BlockSpec((None, 4, Fp), lambda b: (b, 0, 0))
    out3_spec = pl.BlockSpec((None, S, 3), lambda b: (b, 0, 0))
    return pl.pallas_call(
        _tc_body,
        grid=(B,),
        in_specs=[pl.BlockSpec((None, S, 3), lambda b: (b, 0, 0)),
                  face_spec, face_spec, face_spec],
        out_specs=[out3_spec, out3_spec, out3_spec, out3_spec],
        out_shape=[jax.ShapeDtypeStruct((B, S, 3), jnp.float32),
                   jax.ShapeDtypeStruct((B, S, 3), jnp.int32),
                   jax.ShapeDtypeStruct((B, S, 3), jnp.float32),
                   jax.ShapeDtypeStruct((B, S, 3), jnp.float32)],
    )(coords, at, bt, ct)


def _sc_body(iq0_hbm, iq1_hbm, iq2_hbm, cb_hbm, r0_out, r1_out, r2_out,
             iq0_v, iq1_v, iq2_v, r0_v, r1_v, r2_v, sem):
    ppw = iq0_v.shape[0]
    cid = lax.axis_index("c")
    sid = lax.axis_index("s")
    wid = sid * 2 + cid
    base = wid * ppw

    pltpu.sync_copy(iq0_hbm.at[pl.ds(base, ppw)], iq0_v)
    pltpu.sync_copy(iq1_hbm.at[pl.ds(base, ppw)], iq1_v)
    pltpu.sync_copy(iq2_hbm.at[pl.ds(base, ppw)], iq2_v)

    g0 = pltpu.async_copy(cb_hbm.at[iq0_v], r0_v, sem)
    g1 = pltpu.async_copy(cb_hbm.at[iq1_v], r1_v, sem)
    g2 = pltpu.async_copy(cb_hbm.at[iq2_v], r2_v, sem)
    g0.wait()
    g1.wait()
    g2.wait()

    pltpu.sync_copy(r0_v, r0_out.at[pl.ds(base, ppw)])
    pltpu.sync_copy(r1_v, r1_out.at[pl.ds(base, ppw)])
    pltpu.sync_copy(r2_v, r2_out.at[pl.ds(base, ppw)])


def _gather_sc(iq0, iq1, iq2, cb4):
    N = iq0.shape[0]
    W = cb4.shape[1]
    ppw = N // _NW
    mesh = plsc.VectorSubcoreMesh(core_axis_name="c", subcore_axis_name="s")
    row = jax.ShapeDtypeStruct((N, W), jnp.float32)
    kern = functools.partial(
        pl.kernel,
        mesh=mesh,
        out_type=[row, row, row],
        scratch_types=[
            pltpu.VMEM((ppw,), jnp.int32),
            pltpu.VMEM((ppw,), jnp.int32),
            pltpu.VMEM((ppw,), jnp.int32),
            pltpu.VMEM((ppw, W), jnp.float32),
            pltpu.VMEM((ppw, W), jnp.float32),
            pltpu.VMEM((ppw, W), jnp.float32),
            pltpu.SemaphoreType.DMA,
        ],
    )(_sc_body)
    return kern(iq0, iq1, iq2, cb4)


def _wsum_body(r0_ref, r1_ref, r2_ref, md_ref, w_ref, out_ref):
    D = out_ref.shape[1]

    def seg(r_ref, k):
        mk = md_ref[:, k:k + 1]
        out = r_ref[:, 0 * D:1 * D]
        for q in (1, 2, 3):
            out = jnp.where(mk == q, r_ref[:, q * D:(q + 1) * D], out)
        return out

    w0 = w_ref[:, 0:1]
    w1 = w_ref[:, 1:2]
    w2 = w_ref[:, 2:3]
    out_ref[:, :] = (seg(r0_ref, 0) * w0 + seg(r1_ref, 1) * w1) \
        + seg(r2_ref, 2) * w2


def _wsum_tc(r0, r1, r2, mods, w2d):
    N, W = r0.shape
    D = W // 4
    CH = 512
    sup_spec = pl.BlockSpec((CH, W), lambda i: (i, 0))
    tri_spec = pl.BlockSpec((CH, 3), lambda i: (i, 0))
    return pl.pallas_call(
        _wsum_body,
        grid=(N // CH,),
        in_specs=[sup_spec, sup_spec, sup_spec, tri_spec, tri_spec],
        out_specs=pl.BlockSpec((CH, D), lambda i: (i, 0)),
        out_shape=jax.ShapeDtypeStruct((N, D), jnp.float32),
    )(r0, r1, r2, mods, w2d)


def kernel(coords, idx, smpl_V, smpl_F, feature_codebooks):
    B, S, _ = coords.shape
    V = smpl_V.shape[1]
    F = smpl_F.shape[0]
    NS, _, D = feature_codebooks.shape

    sf = smpl_F.astype(jnp.int32)
    a = smpl_V[:, sf[:, 0]]
    b = smpl_V[:, sf[:, 1]]
    c = smpl_V[:, sf[:, 2]]
    Fp = ((F + _FT - 1) // _FT) * _FT

    def face_planes(verts, ids):
        vt = verts.transpose(0, 2, 1)
        idr = jnp.broadcast_to(ids.astype(jnp.float32)[None, None, :],
                               (B, 1, F))
        pl4 = jnp.concatenate([vt, idr], 1)
        pad = jnp.full((B, 4, Fp - F), 1e9, jnp.float32).at[:, 3, :].set(0.0)
        return jnp.concatenate([pl4, pad], 2)

    at = face_planes(a, sf[:, 0])
    bt = face_planes(b, sf[:, 1])
    ct = face_planes(c, sf[:, 2])

    wts, hf, cf, nrm = _closest_point_tc(coords, at, bt, ct)

    rb = jnp.broadcast_to((idx.astype(jnp.int32) * V)[:, None], (B, S))
    gidx = rb[..., None] + hf
    gflat = gidx.reshape(B * S, 3)
    iq = gflat >> 2
    mods = gflat & 3

    cb4 = feature_codebooks.reshape((NS * V) // 4, 4 * D)
    r0, r1, r2 = _gather_sc(iq[:, 0], iq[:, 1], iq[:, 2], cb4)
    wf = _wsum_tc(r0, r1, r2, mods, wts.reshape(B * S, 3))
    return wf.reshape(B, S, D), cf, nrm


# speedup vs baseline: 1.4889x; 1.4889x over previous
import functools

import jax
import jax.numpy as jnp
from jax import lax
from jax.experimental import pallas as pl
from jax.experimental.pallas import tpu as pltpu
from jax.experimental.pallas import tpu_sc as plsc

_FT = 512
_NW = 32


def _safe_div(num, den):
    den = jnp.where(jnp.abs(den) < 1e-12, jnp.where(den < 0, -1e-12, 1e-12), den)
    return num / den


def _tc_body(p_ref, a_ref, b_ref, c_ref, wts_ref, hf_ref, cf_ref, nrm_ref):
    S = p_ref.shape[0]
    Fp = a_ref.shape[1]
    n_tiles = Fp // _FT

    px = p_ref[:, 0:1]
    py = p_ref[:, 1:2]
    pz = p_ref[:, 2:3]

    def tile(t, carry):
        (bd2, bu, bv, bw, bhx, bhy, bhz, bnx, bny, bnz, bf0, bf1, bf2) = carry
        ds = pl.ds(t * _FT, _FT)
        ax = a_ref[0:1, ds]; ay = a_ref[1:2, ds]; az = a_ref[2:3, ds]
        bx = b_ref[0:1, ds]; by = b_ref[1:2, ds]; bz = b_ref[2:3, ds]
        cx = c_ref[0:1, ds]; cy = c_ref[1:2, ds]; cz = c_ref[2:3, ds]
        f0 = a_ref[3:4, ds]; f1 = b_ref[3:4, ds]; f2 = c_ref[3:4, ds]

        abx = bx - ax; aby = by - ay; abz = bz - az
        acx = cx - ax; acy = cy - ay; acz = cz - az

        apx = px - ax; apy = py - ay; apz = pz - az
        d1 = (abx * apx + abz * apz) + aby * apy
        d2 = (acx * apx + acz * apz) + acy * apy
        bpx = px - bx; bpy = py - by; bpz = pz - bz
        d3 = (abx * bpx + abz * bpz) + aby * bpy
        d4 = (acx * bpx + acz * bpz) + acy * bpy
        cpx = px - cx; cpy = py - cy; cpz = pz - cz
        d5 = (abx * cpx + abz * cpz) + aby * cpy
        d6 = (acx * cpx + acz * cpz) + acy * cpy

        vc = d1 * d4 - d3 * d2
        vb = d5 * d2 - d1 * d6
        va = d3 * d6 - d5 * d4
        v_ab = _safe_div(d1, d1 - d3)
        w_ac = _safe_div(d2, d2 - d6)
        w_bc = _safe_div(d4 - d3, (d4 - d3) + (d5 - d6))
        denom = _safe_div(jnp.ones_like(va), va + vb + vc)
        v_in = vb * denom
        w_in = vc * denom
        z = jnp.zeros_like(d1)
        o = jnp.ones_like(d1)
        c1 = (d1 <= 0) & (d2 <= 0)
        c2 = (d3 >= 0) & (d4 <= d3)
        c3 = (vc <= 0) & (d1 >= 0) & (d3 <= 0)
        c4 = (d6 >= 0) & (d5 <= d6)
        c5 = (vb <= 0) & (d2 >= 0) & (d6 <= 0)
        c6 = (va <= 0) & ((d4 - d3) >= 0) & ((d5 - d6) >= 0)
        conds = [c1, c2, c3, c4, c5, c6]

        def _select(vals, default):
            out = default
            for cnd, val in reversed(list(zip(conds, vals))):
                out = jnp.where(cnd, val, out)
            return out

        u = _select([o, z, o - v_ab, z, o - w_ac, z], o - v_in - w_in)
        v = _select([z, o, v_ab, z, z, o - w_bc], v_in)
        w = _select([z, z, z, o, w_ac, w_bc], w_in)

        hx = (u * ax + v * bx) + w * cx
        hy = (u * ay + v * by) + w * cy
        hz = (u * az + v * bz) + w * cz
        dx = hx - px; dy = hy - py; dz = hz - pz
        d2t = (dx * dx + dz * dz) + dy * dy

        nxf = aby * acz - abz * acy
        nyf = abz * acx - abx * acz
        nzf = abx * acy - aby * acx

        m = jnp.min(d2t, axis=1, keepdims=True)
        lane = lax.broadcasted_iota(jnp.int32, (S, _FT), 1)
        lsel = jnp.min(jnp.where(d2t == m, lane, _FT), axis=1, keepdims=True)
        onehot = lane == lsel

        def pick(q):
            return jnp.sum(jnp.where(onehot, q, 0.0), axis=1, keepdims=True)

        tu = pick(u); tv = pick(v); tw = pick(w)
        thx = pick(hx); thy = pick(hy); thz = pick(hz)
        tnx = pick(nxf); tny = pick(nyf); tnz = pick(nzf)
        tf0 = pick(f0); tf1 = pick(f1); tf2 = pick(f2)

        better = m < bd2
        bd2 = jnp.where(better, m, bd2)
        bu = jnp.where(better, tu, bu)
        bv = jnp.where(better, tv, bv)
        bw = jnp.where(better, tw, bw)
        bhx = jnp.where(better, thx, bhx)
        bhy = jnp.where(better, thy, bhy)
        bhz = jnp.where(better, thz, bhz)
        bnx = jnp.where(better, tnx, bnx)
        bny = jnp.where(better, tny, bny)
        bnz = jnp.where(better, tnz, bnz)
        bf0 = jnp.where(better, tf0, bf0)
        bf1 = jnp.where(better, tf1, bf1)
        bf2 = jnp.where(better, tf2, bf2)
        return (bd2, bu, bv, bw, bhx, bhy, bhz, bnx, bny, bnz, bf0, bf1, bf2)

    zf = jnp.zeros((S, 1), jnp.float32)
    init = (jnp.full((S, 1), jnp.inf, jnp.float32),
            zf, zf, zf, zf, zf, zf, zf, zf, zf, zf, zf, zf)
    (bd2, bu, bv, bw, bhx, bhy, bhz, bnx, bny, bnz, bf0, bf1, bf2) = \
        lax.fori_loop(0, n_tiles, tile, init)

    sdot = ((px - bhx) * bnx + (pz - bhz) * bnz) + (py - bhy) * bny
    sgn = jnp.sign(sdot)
    dist = jnp.sqrt(jnp.maximum(bd2, 1e-12))
    sdf = dist * sgn

    dxo = bhx - px; dyo = bhy - py; dzo = bhz - pz
    nrm = jnp.maximum(jnp.sqrt((dxo * dxo + dzo * dzo) + dyo * dyo), 1e-6)

    wts_ref[:, 0:1] = bu
    wts_ref[:, 1:2] = bv
    wts_ref[:, 2:3] = bw
    hf_ref[:, 0:1] = bf0.astype(jnp.int32)
    hf_ref[:, 1:2] = bf1.astype(jnp.int32)
    hf_ref[:, 2:3] = bf2.astype(jnp.int32)
    cf_ref[:, 0:1] = bv
    cf_ref[:, 1:2] = bw
    cf_ref[:, 2:3] = sdf
    nrm_ref[:, 0:1] = dxo / nrm
    nrm_ref[:, 1:2] = dyo / nrm
    nrm_ref[:, 2:3] = dzo / nrm


def _closest_point_tc(coords, at, bt, ct):
    B, S, _ = coords.shape
    Fp = at.shape[2]
    face_spec = pl.BlockSpec((None, 4, Fp), lambda b: (b, 0, 0))
    out3_spec = pl.BlockSpec((None, S, 3), lambda b: (b, 0, 0))
    return pl.pallas_call(
        _tc_body,
        grid=(B,),
        in_specs=[pl.BlockSpec((None, S, 3), lambda b: (b, 0, 0)),
                  face_spec, face_spec, face_spec],
        out_specs=[out3_spec, out3_spec, out3_spec, out3_spec],
        out_shape=[jax.ShapeDtypeStruct((B, S, 3), jnp.float32),
                   jax.ShapeDtypeStruct((B, S, 3), jnp.int32),
                   jax.ShapeDtypeStruct((B, S, 3), jnp.float32),
                   jax.ShapeDtypeStruct((B, S, 3), jnp.float32)],
    )(coords, at, bt, ct)


def _sc_body(iq0_hbm, iq1_hbm, iq2_hbm, cb_hbm, r0_out, r1_out, r2_out,
             iq0_v, iq1_v, iq2_v, r0_v, r1_v, r2_v, sem):
    ppw = iq0_v.shape[0]
    cid = lax.axis_index("c")
    sid = lax.axis_index("s")
    wid = sid * 2 + cid
    base = wid * ppw

    pltpu.sync_copy(iq0_hbm.at[pl.ds(base, ppw)], iq0_v)
    pltpu.sync_copy(iq1_hbm.at[pl.ds(base, ppw)], iq1_v)
    pltpu.sync_copy(iq2_hbm.at[pl.ds(base, ppw)], iq2_v)

    g0 = pltpu.async_copy(cb_hbm.at[iq0_v], r0_v, sem)
    g1 = pltpu.async_copy(cb_hbm.at[iq1_v], r1_v, sem)
    g2 = pltpu.async_copy(cb_hbm.at[iq2_v], r2_v, sem)
    g0.wait()
    g1.wait()
    g2.wait()

    pltpu.sync_copy(r0_v, r0_out.at[pl.ds(base, ppw)])
    pltpu.sync_copy(r1_v, r1_out.at[pl.ds(base, ppw)])
    pltpu.sync_copy(r2_v, r2_out.at[pl.ds(base, ppw)])


def _gather_sc(iq0, iq1, iq2, cb4):
    N = iq0.shape[0]
    W = cb4.shape[1]
    ppw = N // _NW
    mesh = plsc.VectorSubcoreMesh(core_axis_name="c", subcore_axis_name="s")
    row = jax.ShapeDtypeStruct((N, W), jnp.float32)
    kern = functools.partial(
        pl.kernel,
        mesh=mesh,
        out_type=[row, row, row],
        scratch_types=[
            pltpu.VMEM((ppw,), jnp.int32),
            pltpu.VMEM((ppw,), jnp.int32),
            pltpu.VMEM((ppw,), jnp.int32),
            pltpu.VMEM((ppw, W), jnp.float32),
            pltpu.VMEM((ppw, W), jnp.float32),
            pltpu.VMEM((ppw, W), jnp.float32),
            pltpu.SemaphoreType.DMA,
        ],
    )(_sc_body)
    return kern(iq0, iq1, iq2, cb4)


def _wsum_body(r0_ref, r1_ref, r2_ref, md_ref, w_ref, out_ref):
    D = out_ref.shape[1]

    def seg(r_ref, k):
        mk = md_ref[:, k:k + 1]
        out = r_ref[:, 0 * D:1 * D]
        for q in (1, 2, 3):
            out = jnp.where(mk == q, r_ref[:, q * D:(q + 1) * D], out)
        return out

    w0 = w_ref[:, 0:1]
    w1 = w_ref[:, 1:2]
    w2 = w_ref[:, 2:3]
    out_ref[:, :] = (seg(r0_ref, 0) * w0 + seg(r1_ref, 1) * w1) \
        + seg(r2_ref, 2) * w2


def _wsum_tc(r0, r1, r2, mods, w2d):
    N, W = r0.shape
    D = W // 4
    CH = 512
    sup_spec = pl.BlockSpec((CH, W), lambda i: (i, 0))
    tri_spec = pl.BlockSpec((CH, 3), lambda i: (i, 0))
    return pl.pallas_call(
        _wsum_body,
        grid=(N // CH,),
        in_specs=[sup_spec, sup_spec, sup_spec, tri_spec, tri_spec],
        out_specs=pl.BlockSpec((CH, D), lambda i: (i, 0)),
        out_shape=jax.ShapeDtypeStruct((N, D), jnp.float32),
    )(r0, r1, r2, mods, w2d)


def kernel(coords, idx, smpl_V, smpl_F, feature_codebooks):
    B, S, _ = coords.shape
    V = smpl_V.shape[1]
    F = smpl_F.shape[0]
    NS, _, D = feature_codebooks.shape

    sf = smpl_F.astype(jnp.int32)
    a = smpl_V[:, sf[:, 0]]
    b = smpl_V[:, sf[:, 1]]
    c = smpl_V[:, sf[:, 2]]
    Fp = ((F + _FT - 1) // _FT) * _FT

    def face_planes(verts, ids):
        vt = verts.transpose(0, 2, 1)
        idr = jnp.broadcast_to(ids.astype(jnp.float32)[None, None, :],
                               (B, 1, F))
        pl4 = jnp.concatenate([vt, idr], 1)
        pad = jnp.full((B, 4, Fp - F), 1e9, jnp.float32).at[:, 3, :].set(0.0)
        return jnp.concatenate([pl4, pad], 2)

    at = face_planes(a, sf[:, 0])
    bt = face_planes(b, sf[:, 1])
    ct = face_planes(c, sf[:, 2])

    wts, hf, cf, nrm = _closest_point_tc(coords, at, bt, ct)

    rb = jnp.broadcast_to((jnp.arange(B, dtype=jnp.int32) * V)[:, None], (B, S))
    gidx = rb[..., None] + hf
    gflat = gidx.reshape(B * S, 3)
    iq = gflat >> 2
    mods = gflat & 3

    cb_used = feature_codebooks[idx]
    cb4 = cb_used.reshape((B * V) // 4, 4 * D)
    r0, r1, r2 = _gather_sc(iq[:, 0], iq[:, 1], iq[:, 2], cb4)
    wf = _wsum_tc(r0, r1, r2, mods, wts.reshape(B * S, 3))
    return wf.reshape(B, S, D), cf, nrm
